# Initial kernel scaffold; baseline (speedup 1.0000x reference)
import jax
import jax.numpy as jnp
from jax.experimental import pallas as pl


def kernel(adj, ref_pull, ref_push):
    raise NotImplementedError("write your pallas kernel here")


# speedup vs baseline: 1.0558x; 1.0558x over previous
import functools

import jax
import jax.numpy as jnp
from jax.experimental import pallas as pl
from jax.experimental.pallas import tpu as pltpu

THRESH = 0.5
MARGIN = 1.0
WEIGHT = 1.0
PULL_W = 1.0
PUSH_W = 1.0


def _loss_body(adj_ref, pull_ref, push_ref, out_ref, acc_ref, *, g, n, b):
    i = pl.program_id(0)
    inv_nn = 1.0 / (n * n)

    @pl.when(i == 0)
    def _init():
        acc_ref[0] = 0.0

    contrib = 0.0
    for gg in range(g):
        a = (adj_ref[gg] > THRESH).astype(jnp.float32)
        dp = a - pull_ref[0, gg]
        ds = a - push_ref[0, gg]
        pull_mse = jnp.sum(dp * dp) * inv_nn
        push_mse = jnp.sum(ds * ds) * inv_nn
        contrib += PULL_W * pull_mse + PUSH_W * jnp.maximum(MARGIN - push_mse, 0.0)
    acc_ref[0] += WEIGHT * contrib

    @pl.when(i == (b // g) - 1)
    def _fin():
        out_ref[0, 0] = acc_ref[0] * (1.0 / b)


def kernel(adj, ref_pull, ref_push):
    B, N, _ = adj.shape
    S = ref_pull.shape[0]
    G = 8
    grid = (B // G,)
    out = pl.pallas_call(
        functools.partial(_loss_body, g=G, n=N, b=B * S),
        grid=grid,
        in_specs=[
            pl.BlockSpec((G, N, N), lambda i: (i, 0, 0)),
            pl.BlockSpec((1, G, N, N), lambda i: (0, i, 0, 0)),
            pl.BlockSpec((1, G, N, N), lambda i: (0, i, 0, 0)),
        ],
        out_specs=pl.BlockSpec((1, 1), lambda i: (0, 0), memory_space=pltpu.SMEM),
        out_shape=jax.ShapeDtypeStruct((1, 1), jnp.float32),
        scratch_shapes=[pltpu.SMEM((1,), jnp.float32)],
    )(adj, ref_pull, ref_push)
    return out[0, 0]


# speedup vs baseline: 1.0744x; 1.0176x over previous
import functools

import jax
import jax.numpy as jnp
from jax.experimental import pallas as pl
from jax.experimental.pallas import tpu as pltpu

THRESH = 0.5
MARGIN = 1.0
WEIGHT = 1.0
PULL_W = 1.0
PUSH_W = 1.0


def _loss_body(adj_ref, pull_ref, push_ref, out_ref, acc_ref, *, g, n, b):
    i = pl.program_id(0)
    inv_nn = 1.0 / (n * n)

    @pl.when(i == 0)
    def _init():
        acc_ref[0] = 0.0

    contrib = 0.0
    for gg in range(g):
        a = (adj_ref[gg] > THRESH).astype(jnp.float32)
        dp = a - pull_ref[0, gg]
        ds = a - push_ref[0, gg]
        pull_mse = jnp.sum(dp * dp) * inv_nn
        push_mse = jnp.sum(ds * ds) * inv_nn
        contrib += PULL_W * pull_mse + PUSH_W * jnp.maximum(MARGIN - push_mse, 0.0)
    acc_ref[0] += WEIGHT * contrib

    @pl.when(i == (b // g) - 1)
    def _fin():
        out_ref[0, 0] = acc_ref[0] * (1.0 / b)


def kernel(adj, ref_pull, ref_push):
    B, N, _ = adj.shape
    S = ref_pull.shape[0]
    G = 16
    grid = (B // G,)
    out = pl.pallas_call(
        functools.partial(_loss_body, g=G, n=N, b=B * S),
        grid=grid,
        in_specs=[
            pl.BlockSpec((G, N, N), lambda i: (i, 0, 0)),
            pl.BlockSpec((1, G, N, N), lambda i: (0, i, 0, 0)),
            pl.BlockSpec((1, G, N, N), lambda i: (0, i, 0, 0)),
        ],
        out_specs=pl.BlockSpec((1, 1), lambda i: (0, 0), memory_space=pltpu.SMEM),
        out_shape=jax.ShapeDtypeStruct((1, 1), jnp.float32),
        scratch_shapes=[pltpu.SMEM((1,), jnp.float32)],
    )(adj, ref_pull, ref_push)
    return out[0, 0]
